# baseline (speedup 1.0000x reference)
import functools

import jax
import jax.numpy as jnp
from jax import lax
from jax.experimental import pallas as pl
from jax.experimental.pallas import tpu as pltpu, tpu_sc as plsc

_Q = 16384
_K = 4096
_NC = 2
_NS = 16
_NW = _NC * _NS
_QW = _Q // _NW
_L = 16
_G = 4

_sc_mesh = plsc.VectorSubcoreMesh(core_axis_name="c", subcore_axis_name="s")


@functools.partial(
    pl.kernel,
    mesh=_sc_mesh,
    out_type=jax.ShapeDtypeStruct((_Q,), jnp.float32),
    scratch_types=[
        pltpu.VMEM((_QW,), jnp.float32),
        pltpu.VMEM((_QW,), jnp.float32),
        pltpu.VMEM((_K,), jnp.float32),
        pltpu.VMEM((_K,), jnp.float32),
        pltpu.VMEM((_K,), jnp.float32),
        pltpu.VMEM((_QW,), jnp.float32),
    ],
)
def _sc_min_dist(xr_hbm, yr_hbm, bx_hbm, by_hbm, b2_hbm, out_hbm,
                 qx_v, qy_v, bx_v, by_v, b2_v, o_v):
    wid = lax.axis_index("s") * _NC + lax.axis_index("c")
    base = wid * _QW
    pltpu.sync_copy(xr_hbm.at[pl.ds(base, _QW)], qx_v)
    pltpu.sync_copy(yr_hbm.at[pl.ds(base, _QW)], qy_v)
    pltpu.sync_copy(bx_hbm, bx_v)
    pltpu.sync_copy(by_hbm, by_v)
    pltpu.sync_copy(b2_hbm, b2_v)

    lanes = lax.iota(jnp.int32, _L)

    def qblock(blk, carry):
        qbase = blk * _L
        qxb = qx_v[pl.ds(qbase, _L)]
        qyb = qy_v[pl.ds(qbase, _L)]
        res = jnp.zeros((_L,), jnp.float32)
        for sub in range(_L // _G):
            qxs = [qxb[sub * _G + i] for i in range(_G)]
            qys = [qyb[sub * _G + i] for i in range(_G)]
            qxm2 = [-2.0 * v for v in qxs]
            qym2 = [-2.0 * v for v in qys]
            minit = tuple(
                jnp.full((_L,), 3.0e38, jnp.float32) for _ in range(_G))

            def kbody(kj, ms, qxm2=qxm2, qym2=qym2):
                off = kj * _L
                bx = bx_v[pl.ds(off, _L)]
                by = by_v[pl.ds(off, _L)]
                b2 = b2_v[pl.ds(off, _L)]
                return tuple(
                    jnp.minimum(ms[i], qxm2[i] * bx + (qym2[i] * by + b2))
                    for i in range(_G))

            ms = lax.fori_loop(0, _K // _L, kbody, minit, unroll=2)
            for i in range(_G):
                m = ms[i]
                for s in (1, 2, 4, 8):
                    m = jnp.minimum(
                        m, m.at[lanes ^ s].get(mode="promise_in_bounds"))
                md2 = m + (qxs[i] * qxs[i] + qys[i] * qys[i])
                res = jnp.where(lanes == (sub * _G + i), md2, res)
        o_v[pl.ds(qbase, _L)] = res
        return carry

    lax.fori_loop(0, _QW // _L, qblock, 0)
    pltpu.sync_copy(o_v, out_hbm.at[pl.ds(base, _QW)])


@jax.jit
def kernel(x, boundary_points):
    bx = boundary_points[:, 0]
    by = boundary_points[:, 1]
    md2 = _sc_min_dist(x[:, 0], x[:, 1], bx, by, bx * bx + by * by)
    min_dist = jnp.sqrt(jnp.maximum(md2, 0.0))[:, None]
    return jnp.concatenate([x, min_dist], axis=-1)


# speedup vs baseline: 1.0070x; 1.0070x over previous
import functools

import jax
import jax.numpy as jnp
from jax import lax
from jax.experimental import pallas as pl
from jax.experimental.pallas import tpu as pltpu, tpu_sc as plsc

_Q = 16384
_K = 4096
_NC = 2
_NS = 16
_NW = _NC * _NS
_QW = _Q // _NW
_L = 16
_G = 8

_sc_mesh = plsc.VectorSubcoreMesh(core_axis_name="c", subcore_axis_name="s")


@functools.partial(
    pl.kernel,
    mesh=_sc_mesh,
    out_type=jax.ShapeDtypeStruct((_Q,), jnp.float32),
    scratch_types=[
        pltpu.VMEM((_QW,), jnp.float32),
        pltpu.VMEM((_QW,), jnp.float32),
        pltpu.VMEM((_K,), jnp.float32),
        pltpu.VMEM((_K,), jnp.float32),
        pltpu.VMEM((_K,), jnp.float32),
        pltpu.VMEM((_QW,), jnp.float32),
    ],
)
def _sc_min_dist(xr_hbm, yr_hbm, bx_hbm, by_hbm, b2_hbm, out_hbm,
                 qx_v, qy_v, bx_v, by_v, b2_v, o_v):
    wid = lax.axis_index("s") * _NC + lax.axis_index("c")
    base = wid * _QW
    pltpu.sync_copy(xr_hbm.at[pl.ds(base, _QW)], qx_v)
    pltpu.sync_copy(yr_hbm.at[pl.ds(base, _QW)], qy_v)
    pltpu.sync_copy(bx_hbm, bx_v)
    pltpu.sync_copy(by_hbm, by_v)
    pltpu.sync_copy(b2_hbm, b2_v)

    lanes = lax.iota(jnp.int32, _L)

    def qblock(blk, carry):
        qbase = blk * _L
        qxb = qx_v[pl.ds(qbase, _L)]
        qyb = qy_v[pl.ds(qbase, _L)]
        res = jnp.zeros((_L,), jnp.float32)
        for sub in range(_L // _G):
            qxs = [qxb[sub * _G + i] for i in range(_G)]
            qys = [qyb[sub * _G + i] for i in range(_G)]
            qxm2 = [-2.0 * v for v in qxs]
            qym2 = [-2.0 * v for v in qys]
            minit = tuple(
                jnp.full((_L,), 3.0e38, jnp.float32) for _ in range(_G))

            def kbody(kj, ms, qxm2=qxm2, qym2=qym2):
                off = kj * _L
                bx = bx_v[pl.ds(off, _L)]
                by = by_v[pl.ds(off, _L)]
                b2 = b2_v[pl.ds(off, _L)]
                return tuple(
                    jnp.minimum(ms[i], qxm2[i] * bx + (qym2[i] * by + b2))
                    for i in range(_G))

            ms = lax.fori_loop(0, _K // _L, kbody, minit, unroll=2)
            for i in range(_G):
                m = ms[i]
                for s in (1, 2, 4, 8):
                    m = jnp.minimum(
                        m, m.at[lanes ^ s].get(mode="promise_in_bounds"))
                md2 = m + (qxs[i] * qxs[i] + qys[i] * qys[i])
                res = jnp.where(lanes == (sub * _G + i), md2, res)
        o_v[pl.ds(qbase, _L)] = res
        return carry

    lax.fori_loop(0, _QW // _L, qblock, 0)
    pltpu.sync_copy(o_v, out_hbm.at[pl.ds(base, _QW)])


@jax.jit
def kernel(x, boundary_points):
    bx = boundary_points[:, 0]
    by = boundary_points[:, 1]
    md2 = _sc_min_dist(x[:, 0], x[:, 1], bx, by, bx * bx + by * by)
    min_dist = jnp.sqrt(jnp.maximum(md2, 0.0))[:, None]
    return jnp.concatenate([x, min_dist], axis=-1)


# speedup vs baseline: 2.2714x; 2.2555x over previous
import functools

import jax
import jax.numpy as jnp
from jax import lax
from jax.experimental import pallas as pl
from jax.experimental.pallas import tpu as pltpu, tpu_sc as plsc

_Q = 16384
_K = 4096

_QS = 5120
_NC = 2
_NS = 16
_NW = _NC * _NS
_QW = _QS // _NW
_L = 16
_G = 8

_QT = _Q - _QS
_BQ = 1024

_sc_mesh = plsc.VectorSubcoreMesh(core_axis_name="c", subcore_axis_name="s")


@functools.partial(
    pl.kernel,
    mesh=_sc_mesh,
    out_type=jax.ShapeDtypeStruct((_QS,), jnp.float32),
    scratch_types=[
        pltpu.VMEM((_QW,), jnp.float32),
        pltpu.VMEM((_QW,), jnp.float32),
        pltpu.VMEM((_K,), jnp.float32),
        pltpu.VMEM((_K,), jnp.float32),
        pltpu.VMEM((_K,), jnp.float32),
        pltpu.VMEM((_QW,), jnp.float32),
    ],
)
def _sc_min_dist(xr_hbm, yr_hbm, bx_hbm, by_hbm, b2_hbm, out_hbm,
                 qx_v, qy_v, bx_v, by_v, b2_v, o_v):
    wid = lax.axis_index("s") * _NC + lax.axis_index("c")
    base = wid * _QW
    pltpu.sync_copy(xr_hbm.at[pl.ds(base, _QW)], qx_v)
    pltpu.sync_copy(yr_hbm.at[pl.ds(base, _QW)], qy_v)
    pltpu.sync_copy(bx_hbm, bx_v)
    pltpu.sync_copy(by_hbm, by_v)
    pltpu.sync_copy(b2_hbm, b2_v)

    lanes = lax.iota(jnp.int32, _L)

    def qblock(blk, carry):
        qbase = blk * _L
        qxb = qx_v[pl.ds(qbase, _L)]
        qyb = qy_v[pl.ds(qbase, _L)]
        res = jnp.zeros((_L,), jnp.float32)
        for sub in range(_L // _G):
            qxs = [qxb[sub * _G + i] for i in range(_G)]
            qys = [qyb[sub * _G + i] for i in range(_G)]
            qxm2 = [-2.0 * v for v in qxs]
            qym2 = [-2.0 * v for v in qys]
            minit = tuple(
                jnp.full((_L,), 3.0e38, jnp.float32) for _ in range(_G))

            def kbody(kj, ms, qxm2=qxm2, qym2=qym2):
                off = kj * _L
                bx = bx_v[pl.ds(off, _L)]
                by = by_v[pl.ds(off, _L)]
                b2 = b2_v[pl.ds(off, _L)]
                return tuple(
                    jnp.minimum(ms[i], qxm2[i] * bx + (qym2[i] * by + b2))
                    for i in range(_G))

            ms = lax.fori_loop(0, _K // _L, kbody, minit, unroll=2)
            for i in range(_G):
                m = ms[i]
                for s in (1, 2, 4, 8):
                    m = jnp.minimum(
                        m, m.at[lanes ^ s].get(mode="promise_in_bounds"))
                md2 = m + (qxs[i] * qxs[i] + qys[i] * qys[i])
                res = jnp.where(lanes == (sub * _G + i), md2, res)
        o_v[pl.ds(qbase, _L)] = res
        return carry

    lax.fori_loop(0, _QW // _L, qblock, 0)
    pltpu.sync_copy(o_v, out_hbm.at[pl.ds(base, _QW)])


def _tc_min_dist_kernel(x_ref, brow_ref, o_ref):
    xx = x_ref[...]
    qx = xx[:, 0:1]
    qy = xx[:, 1:2]
    qxm2 = -2.0 * qx
    qym2 = -2.0 * qy
    bx = brow_ref[0:1, :]
    by = brow_ref[1:2, :]
    b2 = brow_ref[2:3, :]
    t = (qxm2 * bx + qym2 * by) + b2
    o_ref[...] = jnp.min(t, axis=1, keepdims=True) + (qx * qx + qy * qy)


@jax.jit
def kernel(x, boundary_points):
    bx = boundary_points[:, 0]
    by = boundary_points[:, 1]
    b2 = bx * bx + by * by
    brow = jnp.stack([bx, by, b2])

    md2_sc = _sc_min_dist(x[:_QS, 0], x[:_QS, 1], bx, by, b2)

    md2_tc = pl.pallas_call(
        _tc_min_dist_kernel,
        grid=(_QT // _BQ,),
        in_specs=[
            pl.BlockSpec((_BQ, 2), lambda i: (i, 0)),
            pl.BlockSpec(brow.shape, lambda i: (0, 0)),
        ],
        out_specs=pl.BlockSpec((_BQ, 1), lambda i: (i, 0)),
        out_shape=jax.ShapeDtypeStruct((_QT, 1), x.dtype),
    )(x[_QS:], brow)

    md2 = jnp.concatenate([md2_sc, md2_tc[:, 0]])
    min_dist = jnp.sqrt(jnp.maximum(md2, 0.0))[:, None]
    return jnp.concatenate([x, min_dist], axis=-1)
